# Initial kernel scaffold; baseline (speedup 1.0000x reference)
import jax
import jax.numpy as jnp
from jax.experimental import pallas as pl


def kernel(x, table, W1, b1, W2, b2):
    raise NotImplementedError("write your pallas kernel here")


# speedup vs baseline: 536.2653x; 536.2653x over previous
import jax
import jax.numpy as jnp
from jax.experimental import pallas as pl
from jax.experimental.pallas import tpu as pltpu

VOCAB = 13
L = 200
B = 16384
E = 32
H = 16
O = 4

BLK = 2048


def _histmlp_kernel(xt_ref, table_ref, w1_ref, b1_ref, w2_ref, b2_ref, out_ref):
    xt = xt_ref[...]

    m = jnp.dot(table_ref[...], w1_ref[...].T,
                preferred_element_type=jnp.float32) * (1.0 / L)

    h_pre = jnp.zeros((H, BLK), dtype=jnp.float32)
    m_last = m[VOCAB - 1]
    for v in range(VOCAB - 1):
        eq = (xt == v).astype(jnp.float32)
        s_v = jnp.sum(eq, axis=0)
        mv = (m[v] - m_last)[:, None]
        h_pre = h_pre + mv * s_v[None, :]
    h_pre = h_pre + (L * m_last + b1_ref[0])[:, None]

    h = jnp.maximum(h_pre, 0.0)
    out = jnp.dot(w2_ref[...], h, preferred_element_type=jnp.float32)
    out_ref[...] = out + b2_ref[0][:, None]


def kernel(x, table, W1, b1, W2, b2):
    xt = x.T.astype(jnp.int32)
    out_t = pl.pallas_call(
        _histmlp_kernel,
        grid=(B // BLK,),
        in_specs=[
            pl.BlockSpec((L, BLK), lambda i: (0, i)),
            pl.BlockSpec((VOCAB, E), lambda i: (0, 0)),
            pl.BlockSpec((H, E), lambda i: (0, 0)),
            pl.BlockSpec((1, H), lambda i: (0, 0)),
            pl.BlockSpec((O, H), lambda i: (0, 0)),
            pl.BlockSpec((1, O), lambda i: (0, 0)),
        ],
        out_specs=pl.BlockSpec((O, BLK), lambda i: (0, i)),
        out_shape=jax.ShapeDtypeStruct((O, B), jnp.float32),
        compiler_params=pltpu.CompilerParams(
            dimension_semantics=("arbitrary",),
        ),
    )(xt, table, W1, b1.reshape(1, H), W2, b2.reshape(1, O))
    return out_t.T
